# baseline (device time: 22273 ns/iter reference)
import jax
import jax.numpy as jnp
from jax import lax
from jax.experimental import pallas as pl
from jax.experimental.pallas import tpu as pltpu

N_DEV = 8
N_EXP_LOCAL = 2
CAP = 64
SLOTS = CAP * N_EXP_LOCAL


def kernel(x, router_W, route_idx, expert_W, shared_W):
    n_tok, d_model = x.shape
    n_exp_local, _, d_hidden = expert_W.shape
    n_exp = router_W.shape[1]

    def body(x_ref, router_ref, route_ref, expw_ref, sharedw_ref,
             out_ref, a_ref, b_ref, rstage_ref, r_ref,
             disp_send_sems, disp_recv_sems, ret_send_sems, ret_recv_sems):
        my = lax.axis_index("i")

        barrier_sem = pltpu.get_barrier_semaphore()
        for d in range(1, N_DEV):
            pl.semaphore_signal(
                barrier_sem, inc=1,
                device_id=(lax.rem(my + d, N_DEV),),
                device_id_type=pl.DeviceIdType.MESH,
            )

        x32 = x_ref[...]
        xb = x32.astype(jnp.bfloat16)
        route = route_ref[...]

        scores = jnp.dot(x32, router_ref[...], preferred_element_type=jnp.float32)
        s_max = jnp.max(scores, axis=-1, keepdims=True)
        e_s = jnp.exp(scores - s_max)
        probs = e_s / jnp.sum(e_s, axis=-1, keepdims=True)
        exp_iota = lax.broadcasted_iota(jnp.int32, (1, n_exp), 1)
        onehot = (route == exp_iota).astype(jnp.float32)
        g_tok = jnp.sum(probs * onehot, axis=1, keepdims=True)
        xg = (x32 * g_tok).astype(jnp.bfloat16)

        iota_r = lax.broadcasted_iota(jnp.int32, (n_tok, n_tok), 0)
        iota_c = lax.broadcasted_iota(jnp.int32, (n_tok, n_tok), 1)
        l_strict = (iota_r > iota_c).astype(jnp.float32)
        rank_all = jnp.dot(l_strict, onehot,
                           preferred_element_type=jnp.float32)

        slot_iota = lax.broadcasted_iota(jnp.int32, (1, SLOTS), 1)
        slot_k = slot_iota // CAP
        slot_c = slot_iota % CAP

        dcols = []
        for d in range(N_DEV):
            e_row = d * N_EXP_LOCAL + slot_k
            e_sel = (lax.broadcasted_iota(jnp.int32, (n_exp, SLOTS), 0)
                     == e_row).astype(jnp.float32)
            rank_sel = jnp.dot(rank_all, e_sel,
                               preferred_element_type=jnp.float32)
            dd = ((route == e_row)
                  & (rank_sel.astype(jnp.int32) == slot_c)
                  ).astype(jnp.bfloat16)
            dcols.append(dd)
            a_blk = lax.dot_general(
                dd, xg, (((0,), (0,)), ((), ())),
                preferred_element_type=jnp.float32)
            a_ref[d] = a_blk.astype(jnp.bfloat16)

        pl.semaphore_wait(barrier_sem, N_DEV - 1)

        for d in range(N_DEV):
            @pl.when(my != d)
            def _(d=d):
                rd = pltpu.make_async_remote_copy(
                    src_ref=a_ref.at[d],
                    dst_ref=b_ref.at[my],
                    send_sem=disp_send_sems.at[d],
                    recv_sem=disp_recv_sems.at[my],
                    device_id=(d,),
                    device_id_type=pl.DeviceIdType.MESH,
                )
                rd.start()

            @pl.when(my == d)
            def _(d=d):
                b_ref[d] = a_ref[d]

        acc = jnp.dot(xb, sharedw_ref[...].astype(jnp.bfloat16),
                      preferred_element_type=jnp.float32)

        w0 = expw_ref[0].astype(jnp.bfloat16)
        w1 = expw_ref[1].astype(jnp.bfloat16)
        row_is_k1 = lax.broadcasted_iota(jnp.int32, (SLOTS, 1), 0) >= CAP
        for s in range(N_DEV):
            @pl.when(my != s)
            def _(s=s):
                rd = pltpu.make_async_remote_copy(
                    src_ref=a_ref.at[s],
                    dst_ref=b_ref.at[s],
                    send_sem=disp_send_sems.at[s],
                    recv_sem=disp_recv_sems.at[s],
                    device_id=(s,),
                    device_id_type=pl.DeviceIdType.MESH,
                )
                rd.wait_recv()

            bs = b_ref[s]
            y0 = jnp.dot(bs, w0, preferred_element_type=jnp.float32)
            y1 = jnp.dot(bs, w1, preferred_element_type=jnp.float32)
            rstage_ref[s] = jnp.where(row_is_k1, y1, y0).astype(jnp.bfloat16)

            @pl.when(my != s)
            def _(s=s):
                rr = pltpu.make_async_remote_copy(
                    src_ref=rstage_ref.at[s],
                    dst_ref=r_ref.at[my],
                    send_sem=ret_send_sems.at[s],
                    recv_sem=ret_recv_sems.at[my],
                    device_id=(s,),
                    device_id_type=pl.DeviceIdType.MESH,
                )
                rr.start()

            @pl.when(my == s)
            def _(s=s):
                r_ref[s] = rstage_ref[s]

        for d in range(N_DEV):
            @pl.when(my != d)
            def _(d=d):
                rr = pltpu.make_async_remote_copy(
                    src_ref=rstage_ref.at[d],
                    dst_ref=r_ref.at[d],
                    send_sem=ret_send_sems.at[d],
                    recv_sem=ret_recv_sems.at[d],
                    device_id=(d,),
                    device_id_type=pl.DeviceIdType.MESH,
                )
                rr.wait_recv()

            acc = acc + jnp.dot(dcols[d], r_ref[d],
                                preferred_element_type=jnp.float32)

        out_ref[...] = acc

        for d in range(N_DEV):
            @pl.when(my != d)
            def _(d=d):
                pltpu.make_async_remote_copy(
                    src_ref=a_ref.at[d],
                    dst_ref=b_ref.at[my],
                    send_sem=disp_send_sems.at[d],
                    recv_sem=disp_recv_sems.at[my],
                    device_id=(d,),
                    device_id_type=pl.DeviceIdType.MESH,
                ).wait_send()
                pltpu.make_async_remote_copy(
                    src_ref=rstage_ref.at[d],
                    dst_ref=r_ref.at[my],
                    send_sem=ret_send_sems.at[d],
                    recv_sem=ret_recv_sems.at[my],
                    device_id=(d,),
                    device_id_type=pl.DeviceIdType.MESH,
                ).wait_send()

    return pl.pallas_call(
        body,
        out_shape=jax.ShapeDtypeStruct((n_tok, d_hidden), jnp.float32),
        in_specs=[pl.BlockSpec(memory_space=pltpu.VMEM)] * 5,
        out_specs=pl.BlockSpec(memory_space=pltpu.VMEM),
        scratch_shapes=[
            pltpu.VMEM((N_DEV, SLOTS, d_model), jnp.bfloat16),
            pltpu.VMEM((N_DEV, SLOTS, d_model), jnp.bfloat16),
            pltpu.VMEM((N_DEV, SLOTS, d_hidden), jnp.bfloat16),
            pltpu.VMEM((N_DEV, SLOTS, d_hidden), jnp.bfloat16),
            pltpu.SemaphoreType.DMA((N_DEV,)),
            pltpu.SemaphoreType.DMA((N_DEV,)),
            pltpu.SemaphoreType.DMA((N_DEV,)),
            pltpu.SemaphoreType.DMA((N_DEV,)),
        ],
        compiler_params=pltpu.CompilerParams(collective_id=0),
    )(x, router_W, route_idx, expert_W, shared_W)
